# baseline (speedup 1.0000x reference)
import functools
import math

import jax
import jax.numpy as jnp
from jax import lax
from jax.experimental import pallas as pl
from jax.experimental.pallas import tpu as pltpu
from jax.experimental.pallas import tpu_sc as plsc

EMBED_DIM = 64
EMBED_SCALE = math.sqrt(EMBED_DIM)
CHUNK = 512


@functools.partial(jax.jit, static_argnames=())
def _sc_scaled_gather(idx_flat, weight):
    B = idx_flat.shape[0]
    info = plsc.get_sparse_core_info()
    nw = info.num_cores * info.num_subcores
    b_per_w = B // nw
    n_chunks = b_per_w // CHUNK
    assert b_per_w * nw == B and n_chunks * CHUNK == b_per_w

    mesh = plsc.VectorSubcoreMesh(core_axis_name="c", subcore_axis_name="s")

    @functools.partial(
        pl.kernel,
        mesh=mesh,
        out_type=jax.ShapeDtypeStruct((B, EMBED_DIM), jnp.float32),
        scratch_types=[
            pltpu.VMEM((CHUNK,), jnp.int32),
            pltpu.VMEM((CHUNK, EMBED_DIM), jnp.float32),
            pltpu.SemaphoreType.DMA,
        ],
        compiler_params=pltpu.CompilerParams(use_tc_tiling_on_sc=False),
    )
    def k(idx_hbm, table_hbm, out_hbm, idx_v, rows_v, sem):
        wid = lax.axis_index("s") * info.num_cores + lax.axis_index("c")
        base = wid * b_per_w

        def chunk_body(i, carry):
            off = base + i * CHUNK
            pltpu.sync_copy(idx_hbm.at[pl.ds(off, CHUNK)], idx_v)
            pltpu.async_copy(table_hbm.at[idx_v], rows_v, sem).wait()

            def scale_row(r, c):
                for j in range(EMBED_DIM // 16):
                    sl = pl.ds(j * 16, 16)
                    rows_v[r, sl] = rows_v[r, sl] * EMBED_SCALE
                return c

            lax.fori_loop(0, CHUNK, scale_row, 0)
            pltpu.sync_copy(rows_v, out_hbm.at[pl.ds(off, CHUNK)])
            return carry

        lax.fori_loop(0, n_chunks, chunk_body, 0)

    return k(idx_flat, weight)


def kernel(tokens, weight):
    n, s = tokens.shape
    idx_flat = tokens.reshape(-1).astype(jnp.int32)
    out = _sc_scaled_gather(idx_flat, weight)
    return out.reshape(n, s, EMBED_DIM)


# speedup vs baseline: 1.1330x; 1.1330x over previous
import functools
import math

import jax
import jax.numpy as jnp
from jax import lax
from jax.experimental import pallas as pl
from jax.experimental.pallas import tpu as pltpu
from jax.experimental.pallas import tpu_sc as plsc

EMBED_DIM = 64
EMBED_SCALE = math.sqrt(EMBED_DIM)
CHUNK = 400
NBUF = 2


@jax.jit
def _sc_scaled_gather(idx_flat, weight):
    B = idx_flat.shape[0]
    info = plsc.get_sparse_core_info()
    nw = info.num_cores * info.num_subcores
    b_per_w = B // nw
    n_chunks = b_per_w // CHUNK
    n_super = n_chunks // NBUF
    assert b_per_w * nw == B and n_chunks * CHUNK == b_per_w
    assert n_super * NBUF == n_chunks

    mesh = plsc.VectorSubcoreMesh(core_axis_name="c", subcore_axis_name="s")

    @functools.partial(
        pl.kernel,
        mesh=mesh,
        out_type=jax.ShapeDtypeStruct((B, EMBED_DIM), jnp.float32),
        scratch_types=[
            *[pltpu.VMEM((CHUNK,), jnp.int32) for _ in range(NBUF)],
            *[pltpu.VMEM((CHUNK, EMBED_DIM), jnp.float32) for _ in range(NBUF)],
            *[pltpu.VMEM((CHUNK, EMBED_DIM), jnp.float32) for _ in range(NBUF)],
            *[pltpu.SemaphoreType.DMA for _ in range(2 * NBUF)],
        ],
        compiler_params=pltpu.CompilerParams(use_tc_tiling_on_sc=False),
    )
    def k(idx_hbm, table_hbm, out_hbm, i0, i1, in0, in1, o0, o1, g0, g1, s0, s1):
        idx_v = (i0, i1)
        in_v = (in0, in1)
        out_v = (o0, o1)
        gsem = (g0, g1)
        ssem = (s0, s1)
        wid = lax.axis_index("s") * info.num_cores + lax.axis_index("c")
        base = wid * b_per_w

        def fire_gather(b, chunk_i):
            off = base + chunk_i * CHUNK
            pltpu.sync_copy(idx_hbm.at[pl.ds(off, CHUNK)], idx_v[b])
            pltpu.async_copy(table_hbm.at[idx_v[b]], in_v[b], gsem[b])

        for b in range(NBUF):
            fire_gather(b, b)

        def super_body(g, carry):
            for b in range(NBUF):
                i = g * NBUF + b
                pltpu.make_async_copy(
                    table_hbm.at[idx_v[b]], in_v[b], gsem[b]
                ).wait()

                @pl.when(g > 0)
                def _():
                    pltpu.make_async_copy(
                        out_v[b], out_hbm.at[pl.ds(base, CHUNK)], ssem[b]
                    ).wait()

                @plsc.parallel_loop(0, CHUNK, unroll=4)
                def _(r):
                    for j in range(EMBED_DIM // 16):
                        sl = pl.ds(j * 16, 16)
                        out_v[b][r, sl] = in_v[b][r, sl] * EMBED_SCALE

                off = base + i * CHUNK
                pltpu.async_copy(out_v[b], out_hbm.at[pl.ds(off, CHUNK)], ssem[b])

                @pl.when(i + NBUF < n_chunks)
                def _():
                    fire_gather(b, i + NBUF)
            return carry

        lax.fori_loop(0, n_super, super_body, 0)

        for b in range(NBUF):
            pltpu.make_async_copy(
                out_v[b], out_hbm.at[pl.ds(base, CHUNK)], ssem[b]
            ).wait()

    return k(idx_flat, weight)


def kernel(tokens, weight):
    n, s = tokens.shape
    idx_flat = tokens.reshape(-1).astype(jnp.int32)
    out = _sc_scaled_gather(idx_flat, weight)
    return out.reshape(n, s, EMBED_DIM)
